# baseline (speedup 1.0000x reference)
import functools

import jax
import jax.numpy as jnp
from jax import lax
from jax.experimental import pallas as pl
from jax.experimental.pallas import tpu as pltpu
from jax.experimental.pallas import tpu_sc as plsc

_N_USER = 5000
_N_ITEM = 5000
_LATDIM = 128
_N_EDGES = 320000
_N_NODES = _N_USER + _N_ITEM

_NC = 2
_NS = 16
_L = 16

_H = _LATDIM // _NC
_HV = _H // _L
_CH = 128
_EPT = 20480
_NCHUNK = _EPT // _CH
_E_PAD = _NS * _EPT
_NP = 10240
_RPT = _NP // _NS
_RB = 128
_NRB = _RPT // _RB
_NB = 4
_NBI = 8
_DG = 2


def _sc_body(xcat, crowsh, valsh, out,
             crows, vals_r, gbuf, xtab, acc1, isem, gsem, ssem):
    xbuf = gbuf.at[0]
    tbuf = gbuf.at[1]
    c = lax.axis_index("c")
    s = lax.axis_index("s")
    base = s * _RPT
    zeros16 = jnp.zeros((_L,), jnp.float32)

    pltpu.sync_copy(xcat.at[pl.ds(c * _NP + base, _RPT)],
                    xtab.at[pl.ds(base, _RPT)])

    def zrow(i, _):
        for d in range(_HV):
            xbuf[i, pl.ds(d * _L, _L)] = zeros16
        return _

    lax.fori_loop(0, _RB, zrow, None)
    for k in range(_NRB):
        pltpu.sync_copy(xbuf, acc1.at[pl.ds(base + k * _RB, _RB)])
    plsc.subcore_barrier()

    def _fire_idx(jd, bd):
        pltpu.async_copy(crowsh.at[s, jd], crows.at[bd], isem.at[bd])
        pltpu.async_copy(valsh.at[s, jd], vals_r.at[bd], isem.at[bd])

    def _wait_idx(jd, bd):
        pltpu.make_async_copy(crowsh.at[s, jd], crows.at[bd], isem.at[bd]).wait()
        pltpu.make_async_copy(valsh.at[s, jd], vals_r.at[bd], isem.at[bd]).wait()

    def _fire_gather(src, bd, bdi):
        pltpu.async_copy(src.at[crows.at[bdi, 0]], gbuf.at[bd], gsem.at[bd])

    def do_layer(src, acc):
        for p in range(_DG + 2):
            _fire_idx(p, p % _NBI)
        for p in range(_DG):
            _wait_idx(p, p % _NBI)
            _fire_gather(src, p % _NB, p % _NBI)

        def chunk_body(j, _):
            jd = j + _DG + 2

            @pl.when(jd < _NCHUNK)
            def _prefetch():
                _fire_idx(jd, lax.rem(jd, _NBI))

            jf = j + _DG

            @pl.when(jf < _NCHUNK)
            def _fire():
                bf = lax.rem(jf, _NB)
                bfi = lax.rem(jf, _NBI)

                @pl.when(jf >= _NB)
                def _drain():
                    pltpu.make_async_copy(
                        gbuf.at[bf],
                        acc.at[crows.at[lax.rem(jf - _NB, _NBI), 1]],
                        ssem.at[bf],
                    ).wait()

                _wait_idx(jf, bfi)
                _fire_gather(src, bf, bfi)

            b = lax.rem(j, _NB)
            bi = lax.rem(j, _NBI)
            pltpu.make_async_copy(
                src.at[crows.at[bi, 0]], gbuf.at[b], gsem.at[b]
            ).wait()

            def scale(gg, _2):
                vv = vals_r[bi, pl.ds(gg * _L, _L)]
                for u in range(_L):
                    v = vv[u]
                    e = gg * _L + u
                    for d in range(_HV):
                        sl = pl.ds(d * _L, _L)
                        gbuf[b, e, sl] = gbuf[b, e, sl] * v
                return _2

            lax.fori_loop(0, _CH // _L, scale, None)
            pltpu.async_copy(
                gbuf.at[b], acc.at[crows.at[bi, 1]], ssem.at[b], add=True
            )
            return _

        lax.fori_loop(0, _NCHUNK, chunk_body, None)
        for m in range(_NCHUNK - (_NB - _DG), _NCHUNK):
            pltpu.make_async_copy(
                gbuf.at[m % _NB], acc.at[crows.at[m % _NBI, 1]],
                ssem.at[m % _NB]
            ).wait()

    do_layer(xtab, acc1)
    plsc.subcore_barrier()

    def addrow(r, _):
        for d in range(_HV):
            sl = pl.ds(d * _L, _L)
            xbuf[r, sl] = xbuf[r, sl] + tbuf[r, sl]
        return _

    for k in range(_NRB):
        r0 = base + k * _RB
        pltpu.sync_copy(xtab.at[pl.ds(r0, _RB)], xbuf)
        pltpu.sync_copy(acc1.at[pl.ds(r0, _RB)], tbuf)
        lax.fori_loop(0, _RB, addrow, None)
        pltpu.sync_copy(xbuf, xtab.at[pl.ds(r0, _RB)])
    plsc.subcore_barrier()

    do_layer(acc1, xtab)
    plsc.subcore_barrier()

    for k in range(_NRB):
        r0 = base + k * _RB
        pltpu.sync_copy(xtab.at[pl.ds(r0, _RB)], xbuf)
        pltpu.sync_copy(xbuf, out.at[pl.ds(c * _NP + r0, _RB)])


@functools.partial(
    pl.kernel,
    out_type=pltpu.HBM((_NC * _NP, _H), jnp.float32),
    mesh=plsc.VectorSubcoreMesh(core_axis_name="c", subcore_axis_name="s"),
    compiler_params=pltpu.CompilerParams(use_tc_tiling_on_sc=False),
    scratch_types=[
        pltpu.VMEM((_NBI, 2, _CH), jnp.int32),
        pltpu.VMEM((_NBI, _CH), jnp.float32),
        pltpu.VMEM((_NB, _CH, _H), jnp.float32),
        pltpu.VMEM_SHARED((_NP, _H), jnp.float32),
        pltpu.VMEM_SHARED((_NP, _H), jnp.float32),
        pltpu.SemaphoreType.DMA((_NBI,)),
        pltpu.SemaphoreType.DMA((_NB,)),
        pltpu.SemaphoreType.DMA((_NB,)),
    ],
)
def _gcn_sc(xcat, crowsh, valsh, out,
            crows, vals_r, gbuf, xtab, acc1, isem, gsem, ssem):
    _sc_body(xcat, crowsh, valsh, out,
             crows, vals_r, gbuf, xtab, acc1, isem, gsem, ssem)


def _merge_body(fin_ref, u_ref, i_ref):
    u_ref[:, :_H] = fin_ref[:_N_USER]
    u_ref[:, _H:] = fin_ref[_NP:_NP + _N_USER]
    i_ref[:, :_H] = fin_ref[_N_USER:_N_NODES]
    i_ref[:, _H:] = fin_ref[_NP + _N_USER:_NP + _N_NODES]


_merge = pl.pallas_call(
    _merge_body,
    out_shape=(
        jax.ShapeDtypeStruct((_N_USER, _LATDIM), jnp.float32),
        jax.ShapeDtypeStruct((_N_ITEM, _LATDIM), jnp.float32),
    ),
)


def kernel(adj_indices, adj_vals, uEmbeds, iEmbeds):
    x = jnp.concatenate([uEmbeds, iEmbeds], axis=0)
    x = jnp.pad(x, ((0, _NP - _N_NODES), (0, 0)))
    xcat = jnp.concatenate([x[:, :_H], x[:, _H:]], axis=0)

    pad = _E_PAD - _N_EDGES
    rows = jnp.pad(adj_indices[0], (0, pad)).reshape(_NS, _NCHUNK, _CH)
    cols = jnp.pad(adj_indices[1], (0, pad)).reshape(_NS, _NCHUNK, _CH)
    crows = jnp.stack([cols, rows], axis=2)
    vals = jnp.pad(adj_vals, (0, pad)).reshape(_NS, _NCHUNK, _CH)

    final = _gcn_sc(xcat, crows, vals)
    return _merge(final)


# speedup vs baseline: 1.5697x; 1.5697x over previous
import functools

import jax
import jax.numpy as jnp
from jax import lax
from jax.experimental import pallas as pl
from jax.experimental.pallas import tpu as pltpu
from jax.experimental.pallas import tpu_sc as plsc

_N_USER = 5000
_N_ITEM = 5000
_LATDIM = 128
_N_EDGES = 320000
_N_NODES = _N_USER + _N_ITEM

_NC = 2
_NS = 16
_L = 16

_H = _LATDIM // _NC
_HV = _H // _L
_CH = 128
_EPT = 20480
_NCHUNK = _EPT // _CH
_E_PAD = _NS * _EPT
_NP = 10240
_RPT = _NP // _NS
_RB = 128
_NRB = _RPT // _RB
_NB = 5
_NBI = 8
_DG = 3


def _sc_body(xcat, crowsh, valsh, out, l1cat,
             crows, vals_r, gbuf, xtab, acc1, isem, gsem, ssem):
    xbuf = gbuf.at[0]
    tbuf = gbuf.at[1]
    c = lax.axis_index("c")
    s = lax.axis_index("s")
    base = s * _RPT
    off = (c * _NP).astype(jnp.int32)
    zeros16 = jnp.zeros((_L,), jnp.float32)

    pltpu.sync_copy(xcat.at[pl.ds(c * _NP + base, _RPT)],
                    xtab.at[pl.ds(base, _RPT)])

    def zrow(i, _):
        for d in range(_HV):
            xbuf[i, pl.ds(d * _L, _L)] = zeros16
        return _

    lax.fori_loop(0, _RB, zrow, None)
    for k in range(_NRB):
        pltpu.sync_copy(xbuf, acc1.at[pl.ds(base + k * _RB, _RB)])
    plsc.subcore_barrier()

    def _fire_idx(jd, bd):
        pltpu.async_copy(crowsh.at[s, jd], crows.at[bd], isem.at[bd])
        pltpu.async_copy(valsh.at[s, jd], vals_r.at[bd], isem.at[bd])

    def _wait_idx(jd, bd):
        pltpu.make_async_copy(crowsh.at[s, jd], crows.at[bd], isem.at[bd]).wait()
        pltpu.make_async_copy(valsh.at[s, jd], vals_r.at[bd], isem.at[bd]).wait()

    def _fire_gather(jj, srcH, srcS, bd, bdi):
        @pl.when(lax.rem(jj, 2) == 0)
        def _h():
            for i in range(_CH // _L):
                sl = pl.ds(i * _L, _L)
                crows[bdi, 0, sl] = crows[bdi, 0, sl] + off
            pltpu.async_copy(srcH.at[crows.at[bdi, 0]], gbuf.at[bd],
                             gsem.at[bd])

        @pl.when(lax.rem(jj, 2) == 1)
        def _s():
            pltpu.async_copy(srcS.at[crows.at[bdi, 0]], gbuf.at[bd],
                             gsem.at[bd])

    def do_layer(srcH, srcS, acc):
        for p in range(_DG + 2):
            _fire_idx(p, p % _NBI)
        for p in range(_DG):
            _wait_idx(p, p % _NBI)
            _fire_gather(p, srcH, srcS, p % _NB, p % _NBI)

        def chunk_body(j, _):
            jd = j + _DG + 2

            @pl.when(jd < _NCHUNK)
            def _prefetch():
                _fire_idx(jd, lax.rem(jd, _NBI))

            jf = j + _DG

            @pl.when(jf < _NCHUNK)
            def _fire():
                bf = lax.rem(jf, _NB)
                bfi = lax.rem(jf, _NBI)

                @pl.when(jf >= _NB)
                def _drain():
                    pltpu.make_async_copy(
                        gbuf.at[bf],
                        acc.at[crows.at[lax.rem(jf - _NB, _NBI), 1]],
                        ssem.at[bf],
                    ).wait()

                _wait_idx(jf, bfi)
                _fire_gather(jf, srcH, srcS, bf, bfi)

            b = lax.rem(j, _NB)
            bi = lax.rem(j, _NBI)
            pltpu.make_async_copy(
                srcS.at[crows.at[bi, 0]], gbuf.at[b], gsem.at[b]
            ).wait()

            def scale(gg, _2):
                vv = vals_r[bi, pl.ds(gg * _L, _L)]
                for u in range(_L):
                    v = vv[u]
                    e = gg * _L + u
                    for d in range(_HV):
                        sl = pl.ds(d * _L, _L)
                        gbuf[b, e, sl] = gbuf[b, e, sl] * v
                return _2

            lax.fori_loop(0, _CH // _L, scale, None)
            pltpu.async_copy(
                gbuf.at[b], acc.at[crows.at[bi, 1]], ssem.at[b], add=True
            )
            return _

        lax.fori_loop(0, _NCHUNK, chunk_body, None)
        for m in range(_NCHUNK - (_NB - _DG), _NCHUNK):
            pltpu.make_async_copy(
                gbuf.at[m % _NB], acc.at[crows.at[m % _NBI, 1]],
                ssem.at[m % _NB]
            ).wait()

    do_layer(xcat, xtab, acc1)
    plsc.subcore_barrier()

    def addrow(r, _):
        for d in range(_HV):
            sl = pl.ds(d * _L, _L)
            xbuf[r, sl] = xbuf[r, sl] + tbuf[r, sl]
        return _

    for k in range(_NRB):
        r0 = base + k * _RB
        pltpu.sync_copy(xtab.at[pl.ds(r0, _RB)], xbuf)
        pltpu.sync_copy(acc1.at[pl.ds(r0, _RB)], tbuf)
        pltpu.sync_copy(tbuf, l1cat.at[pl.ds(c * _NP + r0, _RB)])
        lax.fori_loop(0, _RB, addrow, None)
        pltpu.sync_copy(xbuf, xtab.at[pl.ds(r0, _RB)])
    plsc.subcore_barrier()

    do_layer(l1cat, acc1, xtab)
    plsc.subcore_barrier()

    for k in range(_NRB):
        r0 = base + k * _RB
        pltpu.sync_copy(xtab.at[pl.ds(r0, _RB)], xbuf)
        pltpu.sync_copy(xbuf, out.at[pl.ds(c * _NP + r0, _RB)])


@functools.partial(
    pl.kernel,
    out_type=(
        pltpu.HBM((_NC * _NP, _H), jnp.float32),
        pltpu.HBM((_NC * _NP, _H), jnp.float32),
    ),
    mesh=plsc.VectorSubcoreMesh(core_axis_name="c", subcore_axis_name="s"),
    compiler_params=pltpu.CompilerParams(use_tc_tiling_on_sc=False),
    scratch_types=[
        pltpu.VMEM((_NBI, 2, _CH), jnp.int32),
        pltpu.VMEM((_NBI, _CH), jnp.float32),
        pltpu.VMEM((_NB, _CH, _H), jnp.float32),
        pltpu.VMEM_SHARED((_NP, _H), jnp.float32),
        pltpu.VMEM_SHARED((_NP, _H), jnp.float32),
        pltpu.SemaphoreType.DMA((_NBI,)),
        pltpu.SemaphoreType.DMA((_NB,)),
        pltpu.SemaphoreType.DMA((_NB,)),
    ],
)
def _gcn_sc(xcat, crowsh, valsh, out, l1cat,
            crows, vals_r, gbuf, xtab, acc1, isem, gsem, ssem):
    _sc_body(xcat, crowsh, valsh, out, l1cat,
             crows, vals_r, gbuf, xtab, acc1, isem, gsem, ssem)


def _merge_body(fin_ref, u_ref, i_ref):
    u_ref[:, :_H] = fin_ref[:_N_USER]
    u_ref[:, _H:] = fin_ref[_NP:_NP + _N_USER]
    i_ref[:, :_H] = fin_ref[_N_USER:_N_NODES]
    i_ref[:, _H:] = fin_ref[_NP + _N_USER:_NP + _N_NODES]


_merge = pl.pallas_call(
    _merge_body,
    out_shape=(
        jax.ShapeDtypeStruct((_N_USER, _LATDIM), jnp.float32),
        jax.ShapeDtypeStruct((_N_ITEM, _LATDIM), jnp.float32),
    ),
)


def kernel(adj_indices, adj_vals, uEmbeds, iEmbeds):
    x = jnp.concatenate([uEmbeds, iEmbeds], axis=0)
    x = jnp.pad(x, ((0, _NP - _N_NODES), (0, 0)))
    xcat = jnp.concatenate([x[:, :_H], x[:, _H:]], axis=0)

    pad = _E_PAD - _N_EDGES
    rows = jnp.pad(adj_indices[0], (0, pad)).reshape(_NS, _NCHUNK, _CH)
    cols = jnp.pad(adj_indices[1], (0, pad)).reshape(_NS, _NCHUNK, _CH)
    crows = jnp.stack([cols, rows], axis=2)
    vals = jnp.pad(adj_vals, (0, pad)).reshape(_NS, _NCHUNK, _CH)

    final, _l1 = _gcn_sc(xcat, crows, vals)
    return _merge(final)


# speedup vs baseline: 1.6622x; 1.0589x over previous
import functools

import jax
import jax.numpy as jnp
from jax import lax
from jax.experimental import pallas as pl
from jax.experimental.pallas import tpu as pltpu
from jax.experimental.pallas import tpu_sc as plsc

_N_USER = 5000
_N_ITEM = 5000
_LATDIM = 128
_N_EDGES = 320000
_N_NODES = _N_USER + _N_ITEM

_NC = 2
_NS = 16
_L = 16

_H = _LATDIM // _NC
_HV = _H // _L
_CH = 128
_EPT = 20480
_NCHUNK = _EPT // _CH
_E_PAD = _NS * _EPT
_NP = 10240
_RPT = _NP // _NS
_RB = 128
_NRB = _RPT // _RB
_NB = 5
_NBI = 8
_DG = 3


def _sc_body(xcat, crowsh, valsh, out, l1cat,
             crows, vals_r, gbuf, xtab, acc1, isem, gsem, ssem):
    xbuf = gbuf.at[0]
    tbuf = gbuf.at[1]
    c = lax.axis_index("c")
    s = lax.axis_index("s")
    base = s * _RPT
    off = (c * _NP).astype(jnp.int32)
    zeros16 = jnp.zeros((_L,), jnp.float32)

    pltpu.sync_copy(xcat.at[pl.ds(c * _NP + base, _RPT)],
                    xtab.at[pl.ds(base, _RPT)])

    def zrow(i, _):
        for d in range(_HV):
            xbuf[i, pl.ds(d * _L, _L)] = zeros16
        return _

    lax.fori_loop(0, _RB, zrow, None)
    for k in range(_NRB):
        pltpu.sync_copy(xbuf, acc1.at[pl.ds(base + k * _RB, _RB)])
    plsc.subcore_barrier()

    def _fire_idx(jd, bd):
        pltpu.async_copy(crowsh.at[s, jd], crows.at[bd], isem.at[bd])
        pltpu.async_copy(valsh.at[s, jd], vals_r.at[bd], isem.at[bd])

    def _wait_idx(jd, bd):
        pltpu.make_async_copy(crowsh.at[s, jd], crows.at[bd], isem.at[bd]).wait()
        pltpu.make_async_copy(valsh.at[s, jd], vals_r.at[bd], isem.at[bd]).wait()

    def _fire_gather(jj, srcH, srcS, bd, bdi):
        @pl.when(lax.rem(jj, 2) == 0)
        def _h():
            for i in range(_CH // _L):
                sl = pl.ds(i * _L, _L)
                crows[bdi, 0, sl] = crows[bdi, 0, sl] + off
            pltpu.async_copy(srcH.at[crows.at[bdi, 0]], gbuf.at[bd],
                             gsem.at[bd])

        @pl.when(lax.rem(jj, 2) == 1)
        def _s():
            pltpu.async_copy(srcS.at[crows.at[bdi, 0]], gbuf.at[bd],
                             gsem.at[bd])

    def do_layer(srcH, srcS, acc):
        for p in range(_DG + 2):
            _fire_idx(p, p % _NBI)
        for p in range(_DG):
            _wait_idx(p, p % _NBI)
            _fire_gather(p, srcH, srcS, p % _NB, p % _NBI)

        def chunk_body(j, _):
            jd = j + _DG + 2

            @pl.when(jd < _NCHUNK)
            def _prefetch():
                _fire_idx(jd, lax.rem(jd, _NBI))

            jf = j + _DG

            @pl.when(jf < _NCHUNK)
            def _fire():
                bf = lax.rem(jf, _NB)
                bfi = lax.rem(jf, _NBI)

                @pl.when(jf >= _NB)
                def _drain():
                    pltpu.make_async_copy(
                        gbuf.at[bf],
                        acc.at[crows.at[lax.rem(jf - _NB, _NBI), 1]],
                        ssem.at[bf],
                    ).wait()

                _wait_idx(jf, bfi)
                _fire_gather(jf, srcH, srcS, bf, bfi)

            b = lax.rem(j, _NB)
            bi = lax.rem(j, _NBI)
            pltpu.make_async_copy(
                srcS.at[crows.at[bi, 0]], gbuf.at[b], gsem.at[b]
            ).wait()

            def scale(gg, _2):
                vv = vals_r[bi, pl.ds(gg * _L, _L)]
                for u in range(_L):
                    v = vv[u]
                    e = gg * _L + u
                    for d in range(_HV):
                        sl = pl.ds(d * _L, _L)
                        gbuf[b, e, sl] = gbuf[b, e, sl] * v
                return _2

            lax.fori_loop(0, _CH // _L, scale, None)
            pltpu.async_copy(
                gbuf.at[b], acc.at[crows.at[bi, 1]], ssem.at[b], add=True
            )
            return _

        lax.fori_loop(0, _NCHUNK, chunk_body, None)
        for m in range(_NCHUNK - _NB, _NCHUNK):
            pltpu.make_async_copy(
                gbuf.at[m % _NB], acc.at[crows.at[m % _NBI, 1]],
                ssem.at[m % _NB]
            ).wait()

    do_layer(xcat, xtab, acc1)
    plsc.subcore_barrier()

    def addrow(r, _):
        for d in range(_HV):
            sl = pl.ds(d * _L, _L)
            xbuf[r, sl] = xbuf[r, sl] + tbuf[r, sl]
        return _

    for k in range(_NRB):
        r0 = base + k * _RB
        pltpu.sync_copy(xtab.at[pl.ds(r0, _RB)], xbuf)
        pltpu.sync_copy(acc1.at[pl.ds(r0, _RB)], tbuf)
        pltpu.sync_copy(tbuf, l1cat.at[pl.ds(c * _NP + r0, _RB)])
        lax.fori_loop(0, _RB, addrow, None)
        pltpu.sync_copy(xbuf, xtab.at[pl.ds(r0, _RB)])
    plsc.subcore_barrier()

    do_layer(l1cat, acc1, xtab)
    plsc.subcore_barrier()

    for k in range(_NRB):
        r0 = base + k * _RB
        pltpu.sync_copy(xtab.at[pl.ds(r0, _RB)], xbuf)
        pltpu.sync_copy(xbuf, out.at[pl.ds(c * _NP + r0, _RB)])


@functools.partial(
    pl.kernel,
    out_type=(
        pltpu.HBM((_NC * _NP, _H), jnp.float32),
        pltpu.HBM((_NC * _NP, _H), jnp.float32),
    ),
    mesh=plsc.VectorSubcoreMesh(core_axis_name="c", subcore_axis_name="s"),
    compiler_params=pltpu.CompilerParams(use_tc_tiling_on_sc=False),
    scratch_types=[
        pltpu.VMEM((_NBI, 2, _CH), jnp.int32),
        pltpu.VMEM((_NBI, _CH), jnp.float32),
        pltpu.VMEM((_NB, _CH, _H), jnp.float32),
        pltpu.VMEM_SHARED((_NP, _H), jnp.float32),
        pltpu.VMEM_SHARED((_NP, _H), jnp.float32),
        pltpu.SemaphoreType.DMA((_NBI,)),
        pltpu.SemaphoreType.DMA((_NB,)),
        pltpu.SemaphoreType.DMA((_NB,)),
    ],
)
def _gcn_sc(xcat, crowsh, valsh, out, l1cat,
            crows, vals_r, gbuf, xtab, acc1, isem, gsem, ssem):
    _sc_body(xcat, crowsh, valsh, out, l1cat,
             crows, vals_r, gbuf, xtab, acc1, isem, gsem, ssem)


def _merge_body(fin_ref, u_ref, i_ref):
    u_ref[:, :_H] = fin_ref[:_N_USER]
    u_ref[:, _H:] = fin_ref[_NP:_NP + _N_USER]
    i_ref[:, :_H] = fin_ref[_N_USER:_N_NODES]
    i_ref[:, _H:] = fin_ref[_NP + _N_USER:_NP + _N_NODES]


_merge = pl.pallas_call(
    _merge_body,
    out_shape=(
        jax.ShapeDtypeStruct((_N_USER, _LATDIM), jnp.float32),
        jax.ShapeDtypeStruct((_N_ITEM, _LATDIM), jnp.float32),
    ),
)


def kernel(adj_indices, adj_vals, uEmbeds, iEmbeds):
    x = jnp.concatenate([uEmbeds, iEmbeds], axis=0)
    x = jnp.pad(x, ((0, _NP - _N_NODES), (0, 0)))
    xcat = jnp.concatenate([x[:, :_H], x[:, _H:]], axis=0)

    pad = _E_PAD - _N_EDGES
    rows = jnp.pad(adj_indices[0], (0, pad)).reshape(_NS, _NCHUNK, _CH)
    cols = jnp.pad(adj_indices[1], (0, pad)).reshape(_NS, _NCHUNK, _CH)
    crows = jnp.stack([cols, rows], axis=2)
    vals = jnp.pad(adj_vals, (0, pad)).reshape(_NS, _NCHUNK, _CH)

    final, _l1 = _gcn_sc(xcat, crows, vals)
    return _merge(final)
